# baseline (device time: 193454 ns/iter reference)
import jax
import jax.numpy as jnp
from jax import lax
from jax.experimental import pallas as pl
from jax.experimental.pallas import tpu as pltpu

T = 1024
D = 2048
V_LOC = 16384
TILE = 512
NT = V_LOC // TILE
NEG_BIG = -1e30


def kernel(x, W, labels):
    labels2d = labels.reshape(T, 1)

    def body(x_ref, w_ref, lab_ref, out_ref, part_ref, peer_ref,
             send_sem, recv_sem):
        step = pl.program_id(0)
        my_x = lax.axis_index("x")
        my_y = lax.axis_index("y")
        nbr = (my_x, 1 - my_y)

        @pl.when(step == 0)
        def _():
            barrier_sem = pltpu.get_barrier_semaphore()
            pl.semaphore_signal(
                barrier_sem, inc=1, device_id=nbr,
                device_id_type=pl.DeviceIdType.MESH,
            )
            pl.semaphore_wait(barrier_sem, 1)
            part_ref[:, :] = jnp.zeros((T, 8), jnp.float32)
            part_ref[:, 0:1] = jnp.full((T, 1), NEG_BIG, jnp.float32)

        logits = jnp.dot(x_ref[:, :], w_ref[:, :],
                         preferred_element_type=jnp.float32)

        m_prev = part_ref[:, 0:1]
        s_prev = part_ref[:, 1:2]
        ll_prev = part_ref[:, 2:3]

        tile_m = jnp.max(logits, axis=1, keepdims=True)
        m_new = jnp.maximum(m_prev, tile_m)
        s_new = (s_prev * jnp.exp(m_prev - m_new)
                 + jnp.sum(jnp.exp(logits - m_new), axis=1, keepdims=True))

        local_col = lab_ref[:, 0:1] - my_y * V_LOC - step * TILE
        cols = lax.broadcasted_iota(jnp.int32, (T, TILE), 1)
        hit = cols == local_col
        ll_new = ll_prev + jnp.sum(jnp.where(hit, logits, 0.0),
                                   axis=1, keepdims=True)

        part_ref[:, 0:1] = m_new
        part_ref[:, 1:2] = s_new
        part_ref[:, 2:3] = ll_new

        @pl.when(step == NT - 1)
        def _():
            rdma = pltpu.make_async_remote_copy(
                src_ref=part_ref,
                dst_ref=peer_ref,
                send_sem=send_sem,
                recv_sem=recv_sem,
                device_id=nbr,
                device_id_type=pl.DeviceIdType.MESH,
            )
            rdma.start()
            rdma.wait()

            m_a = part_ref[:, 0:1]
            s_a = part_ref[:, 1:2]
            l_a = part_ref[:, 2:3]
            m_b = peer_ref[:, 0:1]
            s_b = peer_ref[:, 1:2]
            l_b = peer_ref[:, 2:3]
            m = jnp.maximum(m_a, m_b)
            s = s_a * jnp.exp(m_a - m) + s_b * jnp.exp(m_b - m)
            lse = m + jnp.log(s)
            out_ref[:, :] = lse - (l_a + l_b)

    out = pl.pallas_call(
        body,
        grid=(NT,),
        out_shape=jax.ShapeDtypeStruct((T, 1), jnp.float32),
        in_specs=[
            pl.BlockSpec((T, D), lambda i: (0, 0)),
            pl.BlockSpec((D, TILE), lambda i: (0, i)),
            pl.BlockSpec((T, 1), lambda i: (0, 0)),
        ],
        out_specs=pl.BlockSpec((T, 1), lambda i: (0, 0)),
        scratch_shapes=[
            pltpu.VMEM((T, 8), jnp.float32),
            pltpu.VMEM((T, 8), jnp.float32),
            pltpu.SemaphoreType.DMA,
            pltpu.SemaphoreType.DMA,
        ],
        compiler_params=pltpu.CompilerParams(
            dimension_semantics=("arbitrary",),
            collective_id=0,
        ),
    )(x, W, labels2d)
    return out.reshape(T)


# device time: 111651 ns/iter; 1.7327x vs baseline; 1.7327x over previous
import jax
import jax.numpy as jnp
from jax import lax
from jax.experimental import pallas as pl
from jax.experimental.pallas import tpu as pltpu

T = 1024
D = 2048
V_LOC = 16384
TILE = 512
NT = V_LOC // TILE


def kernel(x, W, labels):
    labels2d = labels.reshape(T, 1)

    def body(x_ref, w_ref, lab_ref, out_ref, buf_a, buf_b, s_acc, ll_acc,
             part_ref, peer_ref, send_sem, recv_sem):
        step = pl.program_id(0)
        my_x = lax.axis_index("x")
        my_y = lax.axis_index("y")
        nbr = (my_x, 1 - my_y)
        even = step % 2 == 0

        @pl.when(step == 0)
        def _():
            barrier_sem = pltpu.get_barrier_semaphore()
            pl.semaphore_signal(
                barrier_sem, inc=1, device_id=nbr,
                device_id_type=pl.DeviceIdType.MESH,
            )
            pl.semaphore_wait(barrier_sem, 1)
            s_acc[:, :] = jnp.zeros((T, TILE), jnp.float32)
            ll_acc[:, :] = jnp.zeros((T, TILE), jnp.float32)

        @pl.when((step < NT) & even)
        def _():
            buf_a[:, :] = jnp.dot(x_ref[:, :], w_ref[:, :],
                                  preferred_element_type=jnp.float32)

        @pl.when((step < NT) & jnp.logical_not(even))
        def _():
            buf_b[:, :] = jnp.dot(x_ref[:, :], w_ref[:, :],
                                  preferred_element_type=jnp.float32)

        def process(buf):
            logits = buf[:, :]
            s_acc[:, :] = s_acc[:, :] + jnp.exp(logits)
            local_col = lab_ref[:, 0:1] - my_y * V_LOC - (step - 1) * TILE
            cols = lax.broadcasted_iota(jnp.int32, (T, TILE), 1)
            ll_acc[:, :] = ll_acc[:, :] + jnp.where(
                cols == local_col, logits, 0.0)

        @pl.when((step > 0) & even)
        def _():
            process(buf_b)

        @pl.when((step > 0) & jnp.logical_not(even))
        def _():
            process(buf_a)

        @pl.when(step == NT)
        def _():
            part_ref[:, 0:1] = jnp.sum(s_acc[:, :], axis=1, keepdims=True)
            part_ref[:, 1:2] = jnp.sum(ll_acc[:, :], axis=1, keepdims=True)

            rdma = pltpu.make_async_remote_copy(
                src_ref=part_ref,
                dst_ref=peer_ref,
                send_sem=send_sem,
                recv_sem=recv_sem,
                device_id=nbr,
                device_id_type=pl.DeviceIdType.MESH,
            )
            rdma.start()
            rdma.wait()

            s = part_ref[:, 0:1] + peer_ref[:, 0:1]
            ll = part_ref[:, 1:2] + peer_ref[:, 1:2]
            out_ref[:, :] = jnp.log(s) - ll

    out = pl.pallas_call(
        body,
        grid=(NT + 1,),
        out_shape=jax.ShapeDtypeStruct((T, 1), jnp.float32),
        in_specs=[
            pl.BlockSpec((T, D), lambda i: (0, 0)),
            pl.BlockSpec((D, TILE), lambda i: (0, jnp.minimum(i, NT - 1))),
            pl.BlockSpec((T, 1), lambda i: (0, 0)),
        ],
        out_specs=pl.BlockSpec((T, 1), lambda i: (0, 0)),
        scratch_shapes=[
            pltpu.VMEM((T, TILE), jnp.float32),
            pltpu.VMEM((T, TILE), jnp.float32),
            pltpu.VMEM((T, TILE), jnp.float32),
            pltpu.VMEM((T, TILE), jnp.float32),
            pltpu.VMEM((T, 8), jnp.float32),
            pltpu.VMEM((T, 8), jnp.float32),
            pltpu.SemaphoreType.DMA,
            pltpu.SemaphoreType.DMA,
        ],
        compiler_params=pltpu.CompilerParams(
            dimension_semantics=("arbitrary",),
            collective_id=0,
        ),
    )(x, W, labels2d)
    return out.reshape(T)


# device time: 100591 ns/iter; 1.9232x vs baseline; 1.1100x over previous
import jax
import jax.numpy as jnp
from jax import lax
from jax.experimental import pallas as pl
from jax.experimental.pallas import tpu as pltpu

T = 1024
D = 2048
V_LOC = 16384
TILE = 512
NT = V_LOC // TILE
NS = NT // 2
LANES = 128
NCHUNK = TILE // LANES


def kernel(x, W, labels):
    labels2d = labels.reshape(T, 1)

    def body(x_ref, w_ref, lab_ref, out_ref, buf_a, buf_b, s_acc, ll_acc,
             part_ref, peer_ref, send_sem, recv_sem):
        step = pl.program_id(0)
        my_x = lax.axis_index("x")
        my_y = lax.axis_index("y")
        nbr = (my_x, 1 - my_y)

        @pl.when(step == 0)
        def _():
            barrier_sem = pltpu.get_barrier_semaphore()
            pl.semaphore_signal(
                barrier_sem, inc=1, device_id=nbr,
                device_id_type=pl.DeviceIdType.MESH,
            )
            pl.semaphore_wait(barrier_sem, 1)
            s_acc[:, :] = jnp.zeros((T, LANES), jnp.float32)
            ll_acc[:, :] = jnp.zeros((T, LANES), jnp.float32)
            buf_b[:, :] = jnp.zeros((T, TILE), jnp.float32)

        def process(buf, tile_idx, valid):
            vf = jnp.where(valid, 1.0, 0.0).astype(jnp.float32)
            lc = jnp.where(valid,
                           lab_ref[:, 0:1] - my_y * V_LOC - tile_idx * TILE,
                           -1)
            s = s_acc[:, :]
            l = ll_acc[:, :]
            for k in range(NCHUNK):
                c = buf[:, k * LANES:(k + 1) * LANES]
                s = s + jnp.exp(c) * vf
                cols = (lax.broadcasted_iota(jnp.int32, (T, LANES), 1)
                        + k * LANES)
                l = l + jnp.where(cols == lc, c, 0.0)
            s_acc[:, :] = s
            ll_acc[:, :] = l

        buf_a[:, :] = jnp.dot(x_ref[:, :], w_ref[:, :TILE],
                              preferred_element_type=jnp.float32)
        process(buf_b, 2 * step - 1, step > 0)

        buf_b[:, :] = jnp.dot(x_ref[:, :], w_ref[:, TILE:],
                              preferred_element_type=jnp.float32)
        process(buf_a, 2 * step, step < NS)

        @pl.when(step == NS)
        def _():
            part_ref[:, 0:1] = jnp.sum(s_acc[:, :], axis=1, keepdims=True)
            part_ref[:, 1:2] = jnp.sum(ll_acc[:, :], axis=1, keepdims=True)

            rdma = pltpu.make_async_remote_copy(
                src_ref=part_ref,
                dst_ref=peer_ref,
                send_sem=send_sem,
                recv_sem=recv_sem,
                device_id=nbr,
                device_id_type=pl.DeviceIdType.MESH,
            )
            rdma.start()
            rdma.wait()

            s = part_ref[:, 0:1] + peer_ref[:, 0:1]
            ll = part_ref[:, 1:2] + peer_ref[:, 1:2]
            out_ref[:, :] = jnp.log(s) - ll

    out = pl.pallas_call(
        body,
        grid=(NS + 1,),
        out_shape=jax.ShapeDtypeStruct((T, 1), jnp.float32),
        in_specs=[
            pl.BlockSpec((T, D), lambda i: (0, 0)),
            pl.BlockSpec((D, 2 * TILE), lambda i: (0, jnp.minimum(i, NS - 1))),
            pl.BlockSpec((T, 1), lambda i: (0, 0)),
        ],
        out_specs=pl.BlockSpec((T, 1), lambda i: (0, 0)),
        scratch_shapes=[
            pltpu.VMEM((T, TILE), jnp.float32),
            pltpu.VMEM((T, TILE), jnp.float32),
            pltpu.VMEM((T, LANES), jnp.float32),
            pltpu.VMEM((T, LANES), jnp.float32),
            pltpu.VMEM((T, 8), jnp.float32),
            pltpu.VMEM((T, 8), jnp.float32),
            pltpu.SemaphoreType.DMA,
            pltpu.SemaphoreType.DMA,
        ],
        compiler_params=pltpu.CompilerParams(
            dimension_semantics=("arbitrary",),
            collective_id=0,
        ),
    )(x, W, labels2d)
    return out.reshape(T)


# device time: 69968 ns/iter; 2.7649x vs baseline; 1.4377x over previous
import jax
import jax.numpy as jnp
from jax import lax
from jax.experimental import pallas as pl
from jax.experimental.pallas import tpu as pltpu

T = 1024
D = 2048
V_LOC = 16384
V_EFF = 8192
TILE = 512
BLK = 2 * TILE
NB = V_EFF // BLK
NS = NB
LANES = 128
NCHUNK = TILE // LANES


def kernel(x, W, labels):
    labels2d = labels.reshape(T, 1)

    def body(x_ref, w_hbm, lab_ref, out_ref, w_a, w_b, buf_a, buf_b,
             s_acc, ll_acc, part_ref, peer_y, peer_x,
             wsem, send_y, recv_y, send_x, recv_x):
        step = pl.program_id(0)
        my_x = lax.axis_index("x")
        my_y = lax.axis_index("y")
        nbr_y = (my_x, 1 - my_y)
        nbr_x = (1 - my_x, my_y)
        col0 = my_x * V_EFF
        even = step % 2 == 0

        def w_dma(blk, wbuf, sem):
            return pltpu.make_async_copy(
                w_hbm.at[:, pl.ds(col0 + blk * BLK, BLK)], wbuf, sem)

        @pl.when(step == 0)
        def _():
            barrier_sem = pltpu.get_barrier_semaphore()
            for nbr in (nbr_y, nbr_x):
                pl.semaphore_signal(
                    barrier_sem, inc=1, device_id=nbr,
                    device_id_type=pl.DeviceIdType.MESH,
                )
            pl.semaphore_wait(barrier_sem, 2)
            w_dma(0, w_a, wsem.at[0]).start()
            w_dma(1, w_b, wsem.at[1]).start()
            s_acc[:, :] = jnp.zeros((T, LANES), jnp.float32)
            ll_acc[:, :] = jnp.zeros((T, LANES), jnp.float32)
            buf_b[:, :] = jnp.zeros((T, TILE), jnp.float32)

        def process(buf, tile_idx, valid):
            vf = jnp.where(valid, 1.0, 0.0).astype(jnp.float32)
            base = my_y * V_LOC + col0 + tile_idx * TILE
            lc = jnp.where(valid, lab_ref[:, 0:1] - base, -1)
            s = s_acc[:, :]
            l = ll_acc[:, :]
            for k in range(NCHUNK):
                c = buf[:, k * LANES:(k + 1) * LANES]
                s = s + jnp.exp(c) * vf
                cols = (lax.broadcasted_iota(jnp.int32, (T, LANES), 1)
                        + k * LANES)
                l = l + jnp.where(cols == lc, c, 0.0)
            s_acc[:, :] = s
            ll_acc[:, :] = l

        def pair(wbuf, sem):
            w_dma(step, wbuf, sem).wait()
            buf_a[:, :] = jnp.dot(x_ref[:, :], wbuf[:, :TILE],
                                  preferred_element_type=jnp.float32)
            process(buf_b, 2 * step - 1, step > 0)
            buf_b[:, :] = jnp.dot(x_ref[:, :], wbuf[:, TILE:],
                                  preferred_element_type=jnp.float32)
            process(buf_a, 2 * step, True)

            @pl.when(step + 2 < NB)
            def _():
                w_dma(step + 2, wbuf, sem).start()

        @pl.when((step < NS) & even)
        def _():
            pair(w_a, wsem.at[0])

        @pl.when((step < NS) & jnp.logical_not(even))
        def _():
            pair(w_b, wsem.at[1])

        @pl.when(step == NS)
        def _():
            process(buf_b, 2 * NS - 1, True)
            part_ref[:, 0:1] = jnp.sum(s_acc[:, :], axis=1, keepdims=True)
            part_ref[:, 1:2] = jnp.sum(ll_acc[:, :], axis=1, keepdims=True)

            rdma_y = pltpu.make_async_remote_copy(
                src_ref=part_ref,
                dst_ref=peer_y,
                send_sem=send_y,
                recv_sem=recv_y,
                device_id=nbr_y,
                device_id_type=pl.DeviceIdType.MESH,
            )
            rdma_y.start()
            rdma_y.wait()

            part_ref[:, 0:1] = part_ref[:, 0:1] + peer_y[:, 0:1]
            part_ref[:, 1:2] = part_ref[:, 1:2] + peer_y[:, 1:2]

            rdma_x = pltpu.make_async_remote_copy(
                src_ref=part_ref,
                dst_ref=peer_x,
                send_sem=send_x,
                recv_sem=recv_x,
                device_id=nbr_x,
                device_id_type=pl.DeviceIdType.MESH,
            )
            rdma_x.start()
            rdma_x.wait()

            s = part_ref[:, 0:1] + peer_x[:, 0:1]
            ll = part_ref[:, 1:2] + peer_x[:, 1:2]
            out_ref[:, :] = jnp.log(s) - ll

    out = pl.pallas_call(
        body,
        grid=(NS + 1,),
        out_shape=jax.ShapeDtypeStruct((T, 1), jnp.float32),
        in_specs=[
            pl.BlockSpec((T, D), lambda i: (0, 0)),
            pl.BlockSpec(memory_space=pltpu.MemorySpace.HBM),
            pl.BlockSpec((T, 1), lambda i: (0, 0)),
        ],
        out_specs=pl.BlockSpec((T, 1), lambda i: (0, 0)),
        scratch_shapes=[
            pltpu.VMEM((D, BLK), jnp.float32),
            pltpu.VMEM((D, BLK), jnp.float32),
            pltpu.VMEM((T, TILE), jnp.float32),
            pltpu.VMEM((T, TILE), jnp.float32),
            pltpu.VMEM((T, LANES), jnp.float32),
            pltpu.VMEM((T, LANES), jnp.float32),
            pltpu.VMEM((T, 8), jnp.float32),
            pltpu.VMEM((T, 8), jnp.float32),
            pltpu.VMEM((T, 8), jnp.float32),
            pltpu.SemaphoreType.DMA((2,)),
            pltpu.SemaphoreType.DMA,
            pltpu.SemaphoreType.DMA,
            pltpu.SemaphoreType.DMA,
            pltpu.SemaphoreType.DMA,
        ],
        compiler_params=pltpu.CompilerParams(
            dimension_semantics=("arbitrary",),
            collective_id=0,
            vmem_limit_bytes=100 * 1024 * 1024,
        ),
    )(x, W, labels2d)
    return out.reshape(T)
